# Initial kernel scaffold; baseline (speedup 1.0000x reference)
import jax
import jax.numpy as jnp
from jax import lax
from jax.experimental import pallas as pl
from jax.experimental.pallas import tpu as pltpu
from jax.experimental.pallas import tpu_sc as plsc

_PRIME = 31
_NMIN, _NMAX = 3, 20
_NT = _NMAX - _NMIN + 1
_TBL = 16384
_D = 64
_B, _L = 4, 8192
_NPOS = _B * _L
_NROWS = _NT * (_TBL + 1)
_INV = 1.0 / (_NT + 1)

_NW = 32
_P = 512
_CHUNKS = _NPOS // (_NW * _P)
_G = 512
_SUB = 128


def _hash_body(byte_ref, ids_ref):
    b = byte_ref[...]
    is_dna = (b >= 1) & (b <= 4)
    safe = jnp.where(is_dna, b - 1, 0)
    invalid = jnp.where(is_dna, 0, 1)
    first = jnp.where(
        lax.broadcasted_iota(jnp.int32, b.shape, 1) == 0, 1, 0)
    h = safe
    bad = invalid
    for n in range(2, _NMAX + 1):
        h = (pltpu.roll(h, 1, 1) * _PRIME + safe) & (_TBL - 1)
        bad = pltpu.roll(bad, 1, 1) | first | invalid
        if n >= _NMIN:
            i = n - _NMIN
            ids_ref[i, :, :] = jnp.where(bad == 1, _TBL, h) + i * (_TBL + 1)


_hash_call = pl.pallas_call(
    _hash_body,
    out_shape=jax.ShapeDtypeStruct((_NT, _B, _L), jnp.int32),
)


def _sc_body(tab_ref, ids_ref, out_ref,
             ids_v, out_v0, out_v1, fixpos, gbuf, rows_v, midx, mrows,
             suffix, sem_g, sem_m, sem_i, sem_o0, sem_o1):
    i16 = lax.iota(jnp.int32, 16)
    wid = lax.axis_index("s") * 2 + lax.axis_index("c")

    midx[pl.ds(0, 16)] = jnp.minimum(i16, _NT - 1) * (_TBL + 1) + _TBL
    midx[pl.ds(16, 16)] = jnp.minimum(i16 + 16, _NT - 1) * (_TBL + 1) + _TBL
    pltpu.async_copy(tab_ref.at[midx], mrows, sem_m).wait()

    zf = jnp.zeros((16,), jnp.float32)
    for k in range(4):
        suffix[pl.ds(_NT * _D + 16 * k, 16)] = zf
    for i in range(_NT - 1, -1, -1):
        fi = jnp.full((16,), i, jnp.int32)
        for k in range(4):
            mr = plsc.load_gather(mrows, [fi, i16 + 16 * k])
            suffix[pl.ds(i * _D + 16 * k, 16)] = (
                suffix[pl.ds((i + 1) * _D + 16 * k, 16)] + mr)
    cst = [suffix[pl.ds(16 * k, 16)] * _INV for k in range(4)]

    zi = jnp.zeros((16,), jnp.int32)
    for m in range((_G + 32) // 16):
        gbuf[pl.ds(16 * m, 16)] = zi

    mixed1 = i16 * (_TBL + 1) + _TBL
    rows2 = jnp.minimum(i16 + 16, _NT - 1)
    mixed2 = rows2 * (_TBL + 1) + _TBL
    lane2 = i16 < (_NT - 16)

    def _fix_at(f):
        off = f & (-16)
        lane = f & 15
        v = fixpos[pl.ds(off, 16)]
        return jnp.sum(jnp.where(i16 == lane, v, 0))

    def _cand(jl):
        fullj = jnp.broadcast_to(jl, (16,))
        v1 = plsc.load_gather(ids_v, [i16, fullj])
        m1 = v1 != mixed1
        v2 = plsc.load_gather(ids_v, [rows2, fullj])
        m2 = (v2 != mixed2) & lane2
        return v1, m1, v2, m2

    out_bufs = [out_v0, out_v1]
    out_sems = [sem_o0, sem_o1]
    out_copies = []
    for ch in range(_CHUNKS):
        base = (wid * _CHUNKS + ch) * _P
        out_v = out_bufs[ch % 2]
        pltpu.async_copy(ids_ref.at[:, pl.ds(base, _P)], ids_v, sem_i).wait()

        def scan_body(m, nfix):
            v = plsc.load_gather(ids_v, [zi, i16 + 16 * m])
            msk = v != _TBL
            plsc.store_compressed(fixpos.at[pl.ds(nfix, 16)],
                                  i16 + 16 * m, mask=msk)
            return nfix + jnp.sum(jnp.where(msk, 1, 0))
        nfix = lax.fori_loop(0, _P // 16, scan_body, 0)

        def fill_body(q, carry):
            for u in range(4):
                for k in range(4):
                    out_v[pl.ds(q * 256 + u * 64 + 16 * k, 16)] = cst[k]
            return carry
        lax.fori_loop(0, _P // 4, fill_body, 0)

        def b_body(fstart):
            def p1_body(st):
                f, goff = st
                jl = _fix_at(f)
                v1, m1, v2, m2 = _cand(jl)
                plsc.store_compressed(gbuf.at[pl.ds(goff, 16)], v1, mask=m1)
                c1 = jnp.sum(jnp.where(m1, 1, 0))
                plsc.store_compressed(gbuf.at[pl.ds(goff + c1, 16)], v2,
                                      mask=m2)
                c2 = jnp.sum(jnp.where(m2, 1, 0))
                return f + 1, goff + c1 + c2

            def p1_cond(st):
                f, goff = st
                return (f < nfix) & (goff <= _G - _NT)

            fnext, gtot = lax.while_loop(p1_cond, p1_body, (fstart, 0))

            @pl.when(gtot > 0)
            def _gather():
                @pl.when(gtot <= 32)
                def _g32():
                    pltpu.async_copy(tab_ref.at[gbuf.at[pl.ds(0, 32)]],
                                     rows_v.at[pl.ds(0, 32)], sem_g).wait()

                @pl.when(gtot > 32)
                def _gbig():
                    for t in range(_G // _SUB):
                        @pl.when(gtot > t * _SUB)
                        def _gt(t=t):
                            pltpu.async_copy(
                                tab_ref.at[gbuf.at[pl.ds(t * _SUB, _SUB)]],
                                rows_v.at[pl.ds(t * _SUB, _SUB)],
                                sem_g).wait()

            def p2_body(st):
                f, roff = st
                jl = _fix_at(f)
                v1, m1, v2, m2 = _cand(jl)
                c = jnp.sum(jnp.where(m1, 1, 0)) + jnp.sum(jnp.where(m2, 1, 0))
                accs = tuple(suffix[pl.ds(c * _D + 16 * k, 16)]
                             for k in range(4))

                def inner(r, accs):
                    fr = jnp.broadcast_to(roff + r, (16,))
                    return tuple(
                        a + plsc.load_gather(rows_v, [fr, i16 + 16 * k])
                        for k, a in enumerate(accs))
                accs = lax.fori_loop(0, c, inner, accs)
                for k in range(4):
                    out_v[pl.ds(jl * _D + 16 * k, 16)] = accs[k] * _INV
                return f + 1, roff + c

            lax.while_loop(lambda st: st[0] < fnext, p2_body, (fstart, 0))
            return fnext

        lax.while_loop(lambda f: f < nfix, b_body, 0)

        out_copies.append(
            pltpu.async_copy(out_v, out_ref.at[pl.ds(base * _D, _P * _D)],
                             out_sems[ch % 2]))
    for cp in out_copies:
        cp.wait()


_sc_call = pl.kernel(
    _sc_body,
    out_type=jax.ShapeDtypeStruct((_NPOS * _D,), jnp.float32),
    mesh=plsc.VectorSubcoreMesh(core_axis_name="c", subcore_axis_name="s"),
    scratch_types=[
        pltpu.VMEM((_NT, _P), jnp.int32),
        pltpu.VMEM((_P * _D,), jnp.float32),
        pltpu.VMEM((_P * _D,), jnp.float32),
        pltpu.VMEM((_P + 16,), jnp.int32),
        pltpu.VMEM((_G + 32,), jnp.int32),
        pltpu.VMEM((_G, _D), jnp.float32),
        pltpu.VMEM((32,), jnp.int32),
        pltpu.VMEM((32, _D), jnp.float32),
        pltpu.VMEM(((_NT + 1) * _D,), jnp.float32),
        pltpu.SemaphoreType.DMA,
        pltpu.SemaphoreType.DMA,
        pltpu.SemaphoreType.DMA,
        pltpu.SemaphoreType.DMA,
        pltpu.SemaphoreType.DMA,
    ],
)


def kernel(byte_ids, tables):
    byte_ids = byte_ids.astype(jnp.int32)
    ids = _hash_call(byte_ids).reshape(_NT, _NPOS)
    tab = tables.reshape(_NROWS, _D)
    out = _sc_call(tab, ids)
    return out.reshape(_B, _L, _D).astype(tables.dtype)


# speedup vs baseline: 14.1309x; 14.1309x over previous
import functools

import jax
import jax.numpy as jnp
from jax import lax
from jax.experimental import pallas as pl
from jax.experimental.pallas import tpu as pltpu
from jax.experimental.pallas import tpu_sc as plsc

_PRIME = 31
_NMIN, _NMAX = 3, 20
_NT = _NMAX - _NMIN + 1
_TBL = 16384
_D = 64
_B, _L = 4, 8192
_NPOS = _B * _L
_NROWS = _NT * (_TBL + 1)
_NPAIR = _NROWS * _D // 128
_INV = 1.0 / (_NT + 1)

_NW = 32
_P = 512
_CHUNKS = _NPOS // (_NW * _P)
_G = 256
_SUB = 128


def _hash_body(byte_ref, ids_ref):
    b = byte_ref[...]
    is_dna = (b >= 1) & (b <= 4)
    safe = jnp.where(is_dna, b - 1, 0)
    invalid = jnp.where(is_dna, 0, 1)
    first = jnp.where(
        lax.broadcasted_iota(jnp.int32, b.shape, 1) == 0, 1, 0)
    h = safe
    bad = invalid
    for n in range(2, _NMAX + 1):
        h = (pltpu.roll(h, 1, 1) * _PRIME + safe) & (_TBL - 1)
        bad = pltpu.roll(bad, 1, 1) | first | invalid
        if n >= _NMIN:
            i = n - _NMIN
            ids_ref[i, :, :] = jnp.where(bad == 1, _TBL, h) + i * (_TBL + 1)


_hash_call = pl.pallas_call(
    _hash_body,
    out_shape=jax.ShapeDtypeStruct((_NT, _B, _L), jnp.int32),
)


def _sc_body(tab_ref, ids_ref, out_ref,
             ids_v, out_v0, out_v1, fixpos, gbuf, pbuf, rows_v, midx, mrows,
             suffix, sem_g, sem_m, sem_i, sem_o0, sem_o1):
    i16 = lax.iota(jnp.int32, 16)
    wid = lax.axis_index("s") * 2 + lax.axis_index("c")

    midx[pl.ds(0, 16)] = (jnp.minimum(i16, _NT - 1) * (_TBL + 1) + _TBL) >> 1
    midx[pl.ds(16, 16)] = (
        jnp.minimum(i16 + 16, _NT - 1) * (_TBL + 1) + _TBL) >> 1
    pltpu.async_copy(tab_ref.at[midx], mrows, sem_m).wait()

    zf = jnp.zeros((16,), jnp.float32)
    for k in range(4):
        suffix[pl.ds(_NT * _D + 16 * k, 16)] = zf
    for i in range(_NT - 1, -1, -1):
        half = ((i * (_TBL + 1) + _TBL) & 1) * _D
        for k in range(4):
            mr = mrows[i, pl.ds(half + 16 * k, 16)]
            suffix[pl.ds(i * _D + 16 * k, 16)] = (
                suffix[pl.ds((i + 1) * _D + 16 * k, 16)] + mr)
    cst = [suffix[pl.ds(16 * k, 16)] * _INV for k in range(4)]

    zi = jnp.zeros((16,), jnp.int32)
    for m in range((_G + 32) // 16):
        gbuf[pl.ds(16 * m, 16)] = zi

    mixed1 = i16 * (_TBL + 1) + _TBL
    rows2 = jnp.minimum(i16 + 16, _NT - 1)
    mixed2 = rows2 * (_TBL + 1) + _TBL
    lane2 = i16 < (_NT - 16)

    def _lane(ref, f):
        off = f & (-16)
        lane = f & 15
        v = ref[pl.ds(off, 16)]
        return jnp.sum(jnp.where(i16 == lane, v, 0))

    def _cand(jl):
        fullj = jnp.broadcast_to(jl, (16,))
        v1 = plsc.load_gather(ids_v, [i16, fullj])
        m1 = v1 != mixed1
        v2 = plsc.load_gather(ids_v, [rows2, fullj])
        m2 = (v2 != mixed2) & lane2
        return v1, m1, v2, m2

    out_bufs = [out_v0, out_v1]
    out_sems = [sem_o0, sem_o1]
    out_copies = []
    for ch in range(_CHUNKS):
        base = (wid * _CHUNKS + ch) * _P
        out_v = out_bufs[ch % 2]
        pltpu.async_copy(ids_ref.at[:, pl.ds(base, _P)], ids_v, sem_i).wait()

        def scan_body(m, nfix):
            v = ids_v[0, pl.ds(16 * m, 16)]
            msk = v != _TBL
            plsc.store_compressed(fixpos.at[pl.ds(nfix, 16)],
                                  i16 + 16 * m, mask=msk)
            return nfix + jnp.sum(jnp.where(msk, 1, 0))
        nfix = lax.fori_loop(0, _P // 16, scan_body, 0)

        def fill_body(q, carry):
            for u in range(4):
                for k in range(4):
                    out_v[pl.ds(q * 256 + u * 64 + 16 * k, 16)] = cst[k]
            return carry
        lax.fori_loop(0, _P // 4, fill_body, 0)

        def b_body(fstart):
            def p1_body(st):
                f, goff = st
                jl = _lane(fixpos, f)
                v1, m1, v2, m2 = _cand(jl)
                plsc.store_compressed(gbuf.at[pl.ds(goff, 16)], v1 >> 1,
                                      mask=m1)
                plsc.store_compressed(pbuf.at[pl.ds(goff, 16)],
                                      (v1 & 1) * _D, mask=m1)
                c1 = jnp.sum(jnp.where(m1, 1, 0))
                plsc.store_compressed(gbuf.at[pl.ds(goff + c1, 16)], v2 >> 1,
                                      mask=m2)
                plsc.store_compressed(pbuf.at[pl.ds(goff + c1, 16)],
                                      (v2 & 1) * _D, mask=m2)
                c2 = jnp.sum(jnp.where(m2, 1, 0))
                return f + 1, goff + c1 + c2

            def p1_cond(st):
                f, goff = st
                return (f < nfix) & (goff <= _G - _NT)

            fnext, gtot = lax.while_loop(p1_cond, p1_body, (fstart, 0))

            @pl.when(gtot > 0)
            def _gather():
                @pl.when(gtot <= 32)
                def _g32():
                    pltpu.async_copy(tab_ref.at[gbuf.at[pl.ds(0, 32)]],
                                     rows_v.at[pl.ds(0, 32)], sem_g).wait()

                @pl.when(gtot > 32)
                def _gbig():
                    for t in range(_G // _SUB):
                        @pl.when(gtot > t * _SUB)
                        def _gt(t=t):
                            pltpu.async_copy(
                                tab_ref.at[gbuf.at[pl.ds(t * _SUB, _SUB)]],
                                rows_v.at[pl.ds(t * _SUB, _SUB)],
                                sem_g).wait()

            def p2_body(st):
                f, roff = st
                jl = _lane(fixpos, f)
                v1, m1, v2, m2 = _cand(jl)
                c = jnp.sum(jnp.where(m1, 1, 0)) + jnp.sum(jnp.where(m2, 1, 0))
                accs = tuple(suffix[pl.ds(c * _D + 16 * k, 16)]
                             for k in range(4))

                def inner(r, accs):
                    half = _lane(pbuf, roff + r)
                    return tuple(
                        a + rows_v[roff + r, pl.ds(half + 16 * k, 16)]
                        for k, a in enumerate(accs))
                accs = lax.fori_loop(0, c, inner, accs)
                for k in range(4):
                    out_v[pl.ds(jl * _D + 16 * k, 16)] = accs[k] * _INV
                return f + 1, roff + c

            lax.while_loop(lambda st: st[0] < fnext, p2_body, (fstart, 0))
            return fnext

        lax.while_loop(lambda f: f < nfix, b_body, 0)

        out_copies.append(
            pltpu.async_copy(out_v, out_ref.at[pl.ds(base * _D, _P * _D)],
                             out_sems[ch % 2]))
    for cp in out_copies:
        cp.wait()


@functools.cache
def _sc_call():
    return pl.kernel(
        _sc_body,
        out_type=jax.ShapeDtypeStruct((_NPOS * _D,), jnp.float32),
        mesh=plsc.VectorSubcoreMesh(core_axis_name="c", subcore_axis_name="s"),
        compiler_params=pltpu.CompilerParams(needs_layout_passes=False),
        scratch_types=[
            pltpu.VMEM((_NT, _P), jnp.int32),
            pltpu.VMEM((_P * _D,), jnp.float32),
            pltpu.VMEM((_P * _D,), jnp.float32),
            pltpu.VMEM((_P + 16,), jnp.int32),
            pltpu.VMEM((_G + 32,), jnp.int32),
            pltpu.VMEM((_G + 32,), jnp.int32),
            pltpu.VMEM((_G, 128), jnp.float32),
            pltpu.VMEM((32,), jnp.int32),
            pltpu.VMEM((32, 128), jnp.float32),
            pltpu.VMEM(((_NT + 1) * _D,), jnp.float32),
            pltpu.SemaphoreType.DMA,
            pltpu.SemaphoreType.DMA,
            pltpu.SemaphoreType.DMA,
            pltpu.SemaphoreType.DMA,
            pltpu.SemaphoreType.DMA,
        ],
    )


def kernel(byte_ids, tables):
    byte_ids = byte_ids.astype(jnp.int32)
    ids = _hash_call(byte_ids).reshape(_NT, _NPOS)
    tab = tables.reshape(_NPAIR, 128)
    out = _sc_call()(tab, ids)
    return out.reshape(_B, _L, _D).astype(tables.dtype)


# speedup vs baseline: 61.3901x; 4.3444x over previous
import functools

import jax
import jax.numpy as jnp
from jax import lax
from jax.experimental import pallas as pl
from jax.experimental.pallas import tpu as pltpu
from jax.experimental.pallas import tpu_sc as plsc

_PRIME = 31
_NMIN, _NMAX = 3, 20
_NT = _NMAX - _NMIN + 1
_TBL = 16384
_STRIDE = 16392
_D = 64
_B, _L = 4, 8192
_NPOS = _B * _L
_INV = 1.0 / (_NT + 1)

_NW = 32
_P = 256
_CHUNKS = _NPOS // (_NW * _P)
_G = 256
_SUB = 128


def _hash_body(byte_ref, ids_ref):
    b = byte_ref[...]
    is_dna = (b >= 1) & (b <= 4)
    safe = jnp.where(is_dna, b - 1, 0)
    invalid = jnp.where(is_dna, 0, 1)
    first = jnp.where(
        lax.broadcasted_iota(jnp.int32, b.shape, 1) == 0, 1, 0)
    h = safe
    bad = invalid
    for n in range(2, _NMAX + 1):
        h = (pltpu.roll(h, 1, 1) * _PRIME + safe) & (_TBL - 1)
        bad = pltpu.roll(bad, 1, 1) | first | invalid
        if n >= _NMIN:
            i = n - _NMIN
            ids_ref[i, :, :] = jnp.where(bad == 1, _TBL, h) + i * _STRIDE


_hash_call = pl.pallas_call(
    _hash_body,
    out_shape=jax.ShapeDtypeStruct((_NT, _B, _L), jnp.int32),
)


def _relayout_body(tab_ref, out_ref):
    out_ref[0, pl.ds(0, _TBL + 1), pl.ds(0, _D)] = tab_ref[0]


_relayout_call = pl.pallas_call(
    _relayout_body,
    grid=(_NT,),
    in_specs=[pl.BlockSpec((1, _TBL + 1, _D), lambda i: (i, 0, 0))],
    out_specs=pl.BlockSpec((1, _STRIDE, 128), lambda i: (i, 0, 0)),
    out_shape=jax.ShapeDtypeStruct((_NT, _STRIDE, 128), jnp.float32),
)


def _sc_body(tab_ref, ids_ref, out_ref,
             ids_v, out_v0, out_v1, fixpos, gbuf, rows_v, midx, mrows,
             suffix, sem_g, sem_m, sem_i, sem_o0, sem_o1):
    i16 = lax.iota(jnp.int32, 16)
    wid = lax.axis_index("s") * 2 + lax.axis_index("c")

    midx[pl.ds(0, 16)] = jnp.minimum(i16, _NT - 1) * _STRIDE + _TBL
    midx[pl.ds(16, 16)] = jnp.minimum(i16 + 16, _NT - 1) * _STRIDE + _TBL
    pltpu.async_copy(tab_ref.at[midx], mrows, sem_m).wait()

    zf = jnp.zeros((16,), jnp.float32)
    for k in range(4):
        suffix[pl.ds(_NT * _D + 16 * k, 16)] = zf
    for i in range(_NT - 1, -1, -1):
        for k in range(4):
            mr = mrows[i, pl.ds(16 * k, 16)]
            suffix[pl.ds(i * _D + 16 * k, 16)] = (
                suffix[pl.ds((i + 1) * _D + 16 * k, 16)] + mr)
    cst = [suffix[pl.ds(16 * k, 16)] * _INV for k in range(4)]

    zi = jnp.zeros((16,), jnp.int32)
    for m in range((_G + 32) // 16):
        gbuf[pl.ds(16 * m, 16)] = zi

    mixed1 = i16 * _STRIDE + _TBL
    rows2 = jnp.minimum(i16 + 16, _NT - 1)
    mixed2 = rows2 * _STRIDE + _TBL
    lane2 = i16 < (_NT - 16)

    def _lane(ref, f):
        off = f & (-16)
        lane = f & 15
        v = ref[pl.ds(off, 16)]
        return jnp.sum(jnp.where(i16 == lane, v, 0))

    def _cand(jl):
        fullj = jnp.broadcast_to(jl, (16,))
        v1 = plsc.load_gather(ids_v, [i16, fullj])
        m1 = v1 != mixed1
        v2 = plsc.load_gather(ids_v, [rows2, fullj])
        m2 = (v2 != mixed2) & lane2
        return v1, m1, v2, m2

    out_bufs = [out_v0, out_v1]
    out_sems = [sem_o0, sem_o1]
    out_copies = []
    for ch in range(_CHUNKS):
        base = (wid * _CHUNKS + ch) * _P
        bb, l0 = base // _L, base % _L
        out_v = out_bufs[ch % 2]
        if ch >= 2:
            out_copies[ch - 2].wait()
        pltpu.async_copy(ids_ref.at[:, bb, pl.ds(l0, _P)], ids_v,
                         sem_i).wait()

        def scan_body(m, nfix):
            v = ids_v[0, pl.ds(16 * m, 16)]
            msk = v != _TBL
            plsc.store_compressed(fixpos.at[pl.ds(nfix, 16)],
                                  i16 + 16 * m, mask=msk)
            return nfix + jnp.sum(jnp.where(msk, 1, 0))
        nfix = lax.fori_loop(0, _P // 16, scan_body, 0)

        def fill_body(q, carry):
            for u in range(4):
                for k in range(4):
                    out_v[q * 4 + u, pl.ds(16 * k, 16)] = cst[k]
            return carry
        lax.fori_loop(0, _P // 4, fill_body, 0)

        def b_body(fstart):
            def p1_body(st):
                f, goff = st
                jl = _lane(fixpos, f)
                v1, m1, v2, m2 = _cand(jl)
                plsc.store_compressed(gbuf.at[pl.ds(goff, 16)], v1, mask=m1)
                c1 = jnp.sum(jnp.where(m1, 1, 0))
                plsc.store_compressed(gbuf.at[pl.ds(goff + c1, 16)], v2,
                                      mask=m2)
                c2 = jnp.sum(jnp.where(m2, 1, 0))
                return f + 1, goff + c1 + c2

            def p1_cond(st):
                f, goff = st
                return (f < nfix) & (goff <= _G - _NT)

            fnext, gtot = lax.while_loop(p1_cond, p1_body, (fstart, 0))

            @pl.when(gtot > 0)
            def _gather():
                @pl.when(gtot <= 32)
                def _g32():
                    pltpu.async_copy(tab_ref.at[gbuf.at[pl.ds(0, 32)]],
                                     rows_v.at[pl.ds(0, 32)], sem_g).wait()

                @pl.when(gtot > 32)
                def _gbig():
                    for t in range(_G // _SUB):
                        @pl.when(gtot > t * _SUB)
                        def _gt(t=t):
                            pltpu.async_copy(
                                tab_ref.at[gbuf.at[pl.ds(t * _SUB, _SUB)]],
                                rows_v.at[pl.ds(t * _SUB, _SUB)],
                                sem_g).wait()

            def p2_body(st):
                f, roff = st
                jl = _lane(fixpos, f)
                v1, m1, v2, m2 = _cand(jl)
                c = jnp.sum(jnp.where(m1, 1, 0)) + jnp.sum(jnp.where(m2, 1, 0))
                accs = tuple(suffix[pl.ds(c * _D + 16 * k, 16)]
                             for k in range(4))

                def inner(r, accs):
                    return tuple(
                        a + rows_v[roff + r, pl.ds(16 * k, 16)]
                        for k, a in enumerate(accs))
                accs = lax.fori_loop(0, c, inner, accs)
                for k in range(4):
                    out_v[jl, pl.ds(16 * k, 16)] = accs[k] * _INV
                return f + 1, roff + c

            lax.while_loop(lambda st: st[0] < fnext, p2_body, (fstart, 0))
            return fnext

        lax.while_loop(lambda f: f < nfix, b_body, 0)

        out_copies.append(
            pltpu.async_copy(out_v, out_ref.at[bb, pl.ds(l0, _P), :],
                             out_sems[ch % 2]))
    for cp in out_copies[-2:]:
        cp.wait()


@functools.cache
def _sc_call():
    return pl.kernel(
        _sc_body,
        out_type=jax.ShapeDtypeStruct((_B, _L, _D), jnp.float32),
        mesh=plsc.VectorSubcoreMesh(core_axis_name="c", subcore_axis_name="s"),
        compiler_params=pltpu.CompilerParams(needs_layout_passes=False),
        scratch_types=[
            pltpu.VMEM((_NT, _P), jnp.int32),
            pltpu.VMEM((_P, _D), jnp.float32),
            pltpu.VMEM((_P, _D), jnp.float32),
            pltpu.VMEM((_P + 16,), jnp.int32),
            pltpu.VMEM((_G + 32,), jnp.int32),
            pltpu.VMEM((_G, 128), jnp.float32),
            pltpu.VMEM((32,), jnp.int32),
            pltpu.VMEM((32, 128), jnp.float32),
            pltpu.VMEM(((_NT + 1) * _D,), jnp.float32),
            pltpu.SemaphoreType.DMA,
            pltpu.SemaphoreType.DMA,
            pltpu.SemaphoreType.DMA,
            pltpu.SemaphoreType.DMA,
            pltpu.SemaphoreType.DMA,
        ],
    )


def kernel(byte_ids, tables):
    byte_ids = byte_ids.astype(jnp.int32)
    ids = _hash_call(byte_ids)
    tab = _relayout_call(tables).reshape(_NT * _STRIDE, 128)
    return _sc_call()(tab, ids).astype(tables.dtype)


# speedup vs baseline: 61.5213x; 1.0021x over previous
import functools

import jax
import jax.numpy as jnp
from jax import lax
from jax.experimental import pallas as pl
from jax.experimental.pallas import tpu as pltpu
from jax.experimental.pallas import tpu_sc as plsc

_PRIME = 31
_NMIN, _NMAX = 3, 20
_NT = _NMAX - _NMIN + 1
_TBL = 16384
_STRIDE = 16392
_D = 64
_B, _L = 4, 8192
_NPOS = _B * _L
_INV = 1.0 / (_NT + 1)

_NW = 32
_P = 256
_CHUNKS = _NPOS // (_NW * _P)
_G = 256
_SUB = 128


def _hash_body(byte_ref, ids_ref):
    b = byte_ref[...]
    is_dna = (b >= 1) & (b <= 4)
    safe = jnp.where(is_dna, b - 1, 0)
    invalid = jnp.where(is_dna, 0, 1)
    first = jnp.where(
        lax.broadcasted_iota(jnp.int32, b.shape, 1) == 0, 1, 0)
    h = safe
    bad = invalid
    for n in range(2, _NMAX + 1):
        h = (pltpu.roll(h, 1, 1) * _PRIME + safe) & (_TBL - 1)
        bad = pltpu.roll(bad, 1, 1) | first | invalid
        if n >= _NMIN:
            i = n - _NMIN
            ids_ref[i, :, :] = jnp.where(bad == 1, _TBL, h) + i * _STRIDE


_hash_call = pl.pallas_call(
    _hash_body,
    out_shape=jax.ShapeDtypeStruct((_NT, _B, _L), jnp.int32),
)


def _relayout_body(tab_ref, out_ref):
    out_ref[0, pl.ds(0, _TBL + 1), pl.ds(0, _D)] = tab_ref[0]


_relayout_call = pl.pallas_call(
    _relayout_body,
    grid=(_NT,),
    in_specs=[pl.BlockSpec((1, _TBL + 1, _D), lambda i: (i, 0, 0))],
    out_specs=pl.BlockSpec((1, _STRIDE, 128), lambda i: (i, 0, 0)),
    out_shape=jax.ShapeDtypeStruct((_NT, _STRIDE, 128), jnp.float32),
)


def _sc_body(tab_ref, ids_ref, out_ref,
             ids_v, out_v0, out_v1, fixpos, gbuf, rows_v, midx, mrows,
             suffix, sem_g, sem_m, sem_i, sem_o0, sem_o1):
    i16 = lax.iota(jnp.int32, 16)
    wid = lax.axis_index("s") * 2 + lax.axis_index("c")

    jax.named_scope
    midx[pl.ds(0, 16)] = jnp.minimum(i16, _NT - 1) * _STRIDE + _TBL
    midx[pl.ds(16, 16)] = jnp.minimum(i16 + 16, _NT - 1) * _STRIDE + _TBL
    pltpu.async_copy(tab_ref.at[midx], mrows, sem_m).wait()

    zf = jnp.zeros((16,), jnp.float32)
    for k in range(4):
        suffix[pl.ds(_NT * _D + 16 * k, 16)] = zf
    for i in range(_NT - 1, -1, -1):
        for k in range(4):
            mr = mrows[i, pl.ds(16 * k, 16)]
            suffix[pl.ds(i * _D + 16 * k, 16)] = (
                suffix[pl.ds((i + 1) * _D + 16 * k, 16)] + mr)
    cst = [suffix[pl.ds(16 * k, 16)] * _INV for k in range(4)]

    zi = jnp.zeros((16,), jnp.int32)
    for m in range((_G + 32) // 16):
        gbuf[pl.ds(16 * m, 16)] = zi

    mixed1 = i16 * _STRIDE + _TBL
    rows2 = jnp.minimum(i16 + 16, _NT - 1)
    mixed2 = rows2 * _STRIDE + _TBL
    lane2 = i16 < (_NT - 16)

    def _lane(ref, f):
        off = f & (-16)
        lane = f & 15
        v = ref[pl.ds(off, 16)]
        return jnp.sum(jnp.where(i16 == lane, v, 0))

    def _cand(jl):
        fullj = jnp.broadcast_to(jl, (16,))
        v1 = plsc.load_gather(ids_v, [i16, fullj])
        m1 = v1 != mixed1
        v2 = plsc.load_gather(ids_v, [rows2, fullj])
        m2 = (v2 != mixed2) & lane2
        return v1, m1, v2, m2

    out_bufs = [out_v0, out_v1]
    out_sems = [sem_o0, sem_o1]
    out_copies = []
    for ch in range(_CHUNKS):
        base = (wid * _CHUNKS + ch) * _P
        bb, l0 = base // _L, base % _L
        out_v = out_bufs[ch % 2]
        if ch >= 2:
            with jax.named_scope("outwait"):
                out_copies[ch - 2].wait()
        with jax.named_scope("idsdma"):
            pltpu.async_copy(ids_ref.at[:, bb, pl.ds(l0, _P)], ids_v,
                             sem_i).wait()

        def scan_body(m, nfix):
            v = ids_v[0, pl.ds(16 * m, 16)]
            msk = v != _TBL
            plsc.store_compressed(fixpos.at[pl.ds(nfix, 16)],
                                  i16 + 16 * m, mask=msk)
            return nfix + jnp.sum(jnp.where(msk, 1, 0))
        with jax.named_scope("scan"):
            nfix = lax.fori_loop(0, _P // 16, scan_body, 0)

        def fill_body(q, carry):
            for u in range(4):
                for k in range(4):
                    out_v[q * 4 + u, pl.ds(16 * k, 16)] = cst[k]
            return carry
        with jax.named_scope("fill"):
            lax.fori_loop(0, _P // 4, fill_body, 0)

        def b_body(fstart):
            def p1_body(st):
                f, goff = st
                jl = _lane(fixpos, f)
                v1, m1, v2, m2 = _cand(jl)
                plsc.store_compressed(gbuf.at[pl.ds(goff, 16)], v1, mask=m1)
                c1 = jnp.sum(jnp.where(m1, 1, 0))
                plsc.store_compressed(gbuf.at[pl.ds(goff + c1, 16)], v2,
                                      mask=m2)
                c2 = jnp.sum(jnp.where(m2, 1, 0))
                return f + 1, goff + c1 + c2

            def p1_cond(st):
                f, goff = st
                return (f < nfix) & (goff <= _G - _NT)

            fnext, gtot = lax.while_loop(p1_cond, p1_body, (fstart, 0))

            @pl.when(gtot > 0)
            def _gather():
                @pl.when(gtot <= 32)
                def _g32():
                    pltpu.async_copy(tab_ref.at[gbuf.at[pl.ds(0, 32)]],
                                     rows_v.at[pl.ds(0, 32)], sem_g).wait()

                @pl.when(gtot > 32)
                def _gbig():
                    for t in range(_G // _SUB):
                        @pl.when(gtot > t * _SUB)
                        def _gt(t=t):
                            pltpu.async_copy(
                                tab_ref.at[gbuf.at[pl.ds(t * _SUB, _SUB)]],
                                rows_v.at[pl.ds(t * _SUB, _SUB)],
                                sem_g).wait()

            def p2_body(st):
                f, roff = st
                jl = _lane(fixpos, f)
                v1, m1, v2, m2 = _cand(jl)
                c = jnp.sum(jnp.where(m1, 1, 0)) + jnp.sum(jnp.where(m2, 1, 0))
                accs = tuple(suffix[pl.ds(c * _D + 16 * k, 16)]
                             for k in range(4))

                def inner(r, accs):
                    return tuple(
                        a + rows_v[roff + r, pl.ds(16 * k, 16)]
                        for k, a in enumerate(accs))
                accs = lax.fori_loop(0, c, inner, accs)
                for k in range(4):
                    out_v[jl, pl.ds(16 * k, 16)] = accs[k] * _INV
                return f + 1, roff + c

            lax.while_loop(lambda st: st[0] < fnext, p2_body, (fstart, 0))
            return fnext

        with jax.named_scope("batches"):
            lax.while_loop(lambda f: f < nfix, b_body, 0)

        out_copies.append(
            pltpu.async_copy(out_v, out_ref.at[bb, pl.ds(l0, _P), :],
                             out_sems[ch % 2]))
    for cp in out_copies[-2:]:
        cp.wait()


@functools.cache
def _sc_call():
    return pl.kernel(
        _sc_body,
        out_type=jax.ShapeDtypeStruct((_B, _L, _D), jnp.float32),
        mesh=plsc.VectorSubcoreMesh(core_axis_name="c", subcore_axis_name="s"),
        compiler_params=pltpu.CompilerParams(needs_layout_passes=False),
        scratch_types=[
            pltpu.VMEM((_NT, _P), jnp.int32),
            pltpu.VMEM((_P, _D), jnp.float32),
            pltpu.VMEM((_P, _D), jnp.float32),
            pltpu.VMEM((_P + 16,), jnp.int32),
            pltpu.VMEM((_G + 32,), jnp.int32),
            pltpu.VMEM((_G, 128), jnp.float32),
            pltpu.VMEM((32,), jnp.int32),
            pltpu.VMEM((32, 128), jnp.float32),
            pltpu.VMEM(((_NT + 1) * _D,), jnp.float32),
            pltpu.SemaphoreType.DMA,
            pltpu.SemaphoreType.DMA,
            pltpu.SemaphoreType.DMA,
            pltpu.SemaphoreType.DMA,
            pltpu.SemaphoreType.DMA,
        ],
    )


def kernel(byte_ids, tables):
    byte_ids = byte_ids.astype(jnp.int32)
    ids = _hash_call(byte_ids)
    tab = _relayout_call(tables).reshape(_NT * _STRIDE, 128)
    return _sc_call()(tab, ids).astype(tables.dtype)
